# baseline (speedup 1.0000x reference)
import functools

import jax
import jax.numpy as jnp
from jax import lax
from jax.experimental import pallas as pl
from jax.experimental.pallas import tpu as pltpu
from jax.experimental.pallas import tpu_sc as plsc

N = 32768
B = 16
F2 = 64
NC = 2
NS = 16
NW = NC * NS

ENC_BLK = 4096
GRID = N // ENC_BLK
ROWS = N // 2
CHUNK = 1024


def _encoder_body(coords_ref, feats_ref, w1a_ref, b1a_ref, w1b_ref, b1b_ref,
                  w2_ref, b2_ref, out_ref, bi_ref):
    x = feats_ref[...]
    cn = (((0,), (0,)), ((), ()))
    h = lax.dot_general(w1a_ref[...], x, cn, preferred_element_type=jnp.float32)
    h = jnp.maximum(h + jnp.transpose(b1a_ref[...]), 0.0)
    h = lax.dot_general(w1b_ref[...], h, cn, preferred_element_type=jnp.float32)
    h = jnp.maximum(h + jnp.transpose(b1b_ref[...]), 0.0)
    w2 = w2_ref[...]
    b2 = b2_ref[...]
    ha = lax.dot_general(h[:, :ENC_BLK // 2], w2, cn,
                         preferred_element_type=jnp.float32)
    hb = lax.dot_general(h[:, ENC_BLK // 2:], w2, cn,
                         preferred_element_type=jnp.float32)
    out_ref[:, :F2] = jnp.maximum(ha + b2, 0.0)
    out_ref[:, F2:] = jnp.maximum(hb + b2, 0.0)
    bi_ref[...] = coords_ref[...][0, :].reshape(ENC_BLK // 128, 128)


def _encoder(coords, feats, W1a, b1a, W1b, b1b, W2, b2):
    full = lambda shape: pl.BlockSpec(shape, lambda i: (0, 0))
    call = pl.pallas_call(
        _encoder_body,
        grid=(GRID,),
        in_specs=[
            pl.BlockSpec((3, ENC_BLK), lambda i: (0, i)),
            pl.BlockSpec((8, ENC_BLK), lambda i: (0, i)),
            full((8, 16)), full((1, 16)),
            full((16, 32)), full((1, 32)),
            full((32, 64)), full((1, 64)),
        ],
        out_specs=(
            pl.BlockSpec((ENC_BLK // 2, 128), lambda i: (i, 0)),
            pl.BlockSpec((ENC_BLK // 128, 128), lambda i: (i, 0)),
        ),
        out_shape=(
            jax.ShapeDtypeStruct((ROWS, 128), jnp.float32),
            jax.ShapeDtypeStruct((N // 128, 128), jnp.int32),
        ),
    )
    ft = (feats.T.reshape(8, GRID, ENC_BLK // 2, 2)
          .transpose(0, 1, 3, 2).reshape(8, N))
    return call(coords.T, ft, W1a, b1a.reshape(1, 16), W1b,
                b1b.reshape(1, 32), W2, b2.reshape(1, 64))


def _pool_body(bi_hbm, f2_hbm, out_hbm, idx_v, rows_v, zer_v, shared):
    c = lax.axis_index("c")
    s = lax.axis_index("s")
    wid = s * NC + c
    q0 = wid * CHUNK

    pltpu.sync_copy(bi_hbm.at[pl.ds(q0, CHUNK)], idx_v)
    pltpu.sync_copy(f2_hbm.at[pl.ds(q0, CHUNK), :], rows_v)

    zero = jnp.zeros((16,), jnp.float32)
    for i in range(B):
        for j in range(F2 // 16):
            zer_v[i, pl.ds(j * 16, 16)] = zero
    pltpu.sync_copy(zer_v, shared.at[pl.ds(s * B, B), :])

    base = s * B
    for g in range(CHUNK // 16):
        idx_v[pl.ds(g * 16, 16)] = idx_v[pl.ds(g * 16, 16)] + base
    pltpu.sync_copy(rows_v, shared.at[idx_v], add=True)

    pltpu.sync_copy(shared.at[pl.ds(s * B, B), :],
                    out_hbm.at[pl.ds(wid * B, B), :])


def _pool(batch_idx_flat, f2_slots):
    mesh = plsc.VectorSubcoreMesh(core_axis_name="c", subcore_axis_name="s")
    f = functools.partial(
        pl.kernel,
        out_type=jax.ShapeDtypeStruct((NW * B, F2), jnp.float32),
        mesh=mesh,
        scratch_types=[
            pltpu.VMEM((CHUNK,), jnp.int32),
            pltpu.VMEM((CHUNK, F2), jnp.float32),
            pltpu.VMEM((B, F2), jnp.float32),
            pltpu.VMEM_SHARED((NS * B, F2), jnp.float32),
        ],
        compiler_params=pltpu.CompilerParams(use_tc_tiling_on_sc=False),
    )(_pool_body)
    return f(batch_idx_flat, f2_slots)


def _head_body(part_ref, bi_ref, wh1_ref, bh1_ref, wh2t_ref, bh2_ref, out_ref):
    x = part_ref[...]
    nr = NW * B // 2
    r = lax.broadcasted_iota(jnp.int32, (B, nr), 1)
    bcol = lax.broadcasted_iota(jnp.int32, (B, nr), 0)
    sel_e = ((2 * r) % B == bcol).astype(jnp.float32)
    sel_o = ((2 * r + 1) % B == bcol).astype(jnp.float32)
    se = jnp.dot(sel_e, x, preferred_element_type=jnp.float32)
    so = jnp.dot(sel_o, x, preferred_element_type=jnp.float32)
    sums = se[:, :F2] + so[:, F2:]
    bi = bi_ref[...]
    counts = [jnp.sum(jnp.where(bi == b, 1.0, 0.0)) for b in range(B)]
    counts = jnp.stack(counts).reshape(B, 1)
    z = sums / jnp.maximum(counts, 1.0)
    h = jnp.dot(z, wh1_ref[...], preferred_element_type=jnp.float32)
    h = jnp.maximum(h + bh1_ref[...], 0.0)
    cn = (((1,), (1,)), ((), ()))
    out_ref[...] = (lax.dot_general(h, wh2t_ref[...], cn,
                                    preferred_element_type=jnp.float32)
                    + bh2_ref[...])


def _head(partials, bi_arr, Wh1, bh1, Wh2, bh2):
    return pl.pallas_call(
        _head_body,
        out_shape=jax.ShapeDtypeStruct((B, 2), jnp.float32),
    )(partials.reshape(NW * B // 2, 128), bi_arr,
      Wh1, bh1.reshape(1, 64), Wh2.T, bh2.reshape(1, 2))


def kernel(coords, feats, W1a, b1a, W1b, b1b, W2, b2, Wh1, bh1, Wh2, bh2):
    f2p, bip = _encoder(coords, feats, W1a, b1a, W1b, b1b, W2, b2)
    partials = _pool(bip.reshape(N), f2p.reshape(N, F2))
    return _head(partials, bip, Wh1, bh1, Wh2, bh2)


# speedup vs baseline: 1.7970x; 1.7970x over previous
import functools

import jax
import jax.numpy as jnp
from jax import lax
from jax.experimental import pallas as pl
from jax.experimental.pallas import tpu as pltpu
from jax.experimental.pallas import tpu_sc as plsc

N = 32768
B = 16
F2 = 64
NC = 2
NS = 16
NW = NC * NS

ENC_BLK = 4096
GRID = N // ENC_BLK
ROWS = N // 2
CHUNK = 1024


def _encoder_body(coords_ref, feats_ref, w1a_ref, b1a_ref, w1b_ref, b1b_ref,
                  w2_ref, b2_ref, out_ref, bi_ref, bip_ref):
    x = feats_ref[...]
    cn = (((0,), (0,)), ((), ()))
    h = lax.dot_general(w1a_ref[...], x, cn, preferred_element_type=jnp.float32)
    h = jnp.maximum(h + jnp.transpose(b1a_ref[...]), 0.0)
    h = lax.dot_general(w1b_ref[...], h, cn, preferred_element_type=jnp.float32)
    h = jnp.maximum(h + jnp.transpose(b1b_ref[...]), 0.0)
    w2 = w2_ref[...]
    b2 = b2_ref[...]
    ha = lax.dot_general(h[:, :ENC_BLK // 2], w2, cn,
                         preferred_element_type=jnp.float32)
    hb = lax.dot_general(h[:, ENC_BLK // 2:], w2, cn,
                         preferred_element_type=jnp.float32)
    out_ref[:, :F2] = jnp.maximum(ha + b2, 0.0)
    out_ref[:, F2:] = jnp.maximum(hb + b2, 0.0)
    bi = coords_ref[...][0, :].reshape(ENC_BLK // 128, 128)
    bi_ref[...] = bi
    shp = (ENC_BLK // 128, 128)
    r = (lax.broadcasted_iota(jnp.int32, shp, 0) * 128
         + lax.broadcasted_iota(jnp.int32, shp, 1))
    wid = (r // (ENC_BLK // 2)) * 16 + 2 * pl.program_id(0) + (
        (r % (ENC_BLK // 2)) // CHUNK)
    bip_ref[...] = bi + (wid // NC) * B


def _encoder(coords, feats, W1a, b1a, W1b, b1b, W2, b2):
    full = lambda shape: pl.BlockSpec(shape, lambda i: (0, 0))
    return pl.pallas_call(
        _encoder_body,
        grid=(GRID,),
        in_specs=[
            pl.BlockSpec((3, ENC_BLK), lambda i: (0, i)),
            pl.BlockSpec((8, ENC_BLK), lambda i: (0, i)),
            full((8, 16)), full((1, 16)),
            full((16, 32)), full((1, 32)),
            full((32, 64)), full((1, 64)),
        ],
        out_specs=(
            pl.BlockSpec((ENC_BLK // 2, 128), lambda i: (i, 0)),
            pl.BlockSpec((ENC_BLK // 128, 128), lambda i: (i, 0)),
            pl.BlockSpec((ENC_BLK // 128, 128), lambda i: (i, 0)),
        ),
        out_shape=(
            jax.ShapeDtypeStruct((ROWS, 128), jnp.float32),
            jax.ShapeDtypeStruct((N // 128, 128), jnp.int32),
            jax.ShapeDtypeStruct((N // 128, 128), jnp.int32),
        ),
    )(coords.T, feats.T, W1a, b1a.reshape(1, 16), W1b, b1b.reshape(1, 32),
      W2, b2.reshape(1, 64))


def _pool_body(bip_hbm, f2_hbm, out_hbm, idx_v, rows_v, zer_v, shared):
    c = lax.axis_index("c")
    s = lax.axis_index("s")
    wid = s * NC + c
    half = wid // 16
    t = wid % 16
    row0 = t * CHUNK
    p0 = (t // 2) * ENC_BLK + half * (ENC_BLK // 2) + (t % 2) * CHUNK

    pltpu.sync_copy(bip_hbm.at[pl.ds(p0, CHUNK)], idx_v)
    pltpu.sync_copy(f2_hbm.at[pl.ds(row0, CHUNK), pl.ds(half * F2, F2)],
                    rows_v)

    zero = jnp.zeros((16,), jnp.float32)
    for i in range(B):
        for j in range(F2 // 16):
            zer_v[i, pl.ds(j * 16, 16)] = zero
    pltpu.sync_copy(zer_v, shared.at[pl.ds(s * B, B), :])

    pltpu.sync_copy(rows_v, shared.at[idx_v], add=True)

    pltpu.sync_copy(shared.at[pl.ds(s * B, B), :],
                    out_hbm.at[pl.ds(wid * B, B), :])


def _pool(batch_idx_flat, f2_rows):
    mesh = plsc.VectorSubcoreMesh(core_axis_name="c", subcore_axis_name="s")
    f = functools.partial(
        pl.kernel,
        out_type=jax.ShapeDtypeStruct((NW * B, F2), jnp.float32),
        mesh=mesh,
        scratch_types=[
            pltpu.VMEM((CHUNK,), jnp.int32),
            pltpu.VMEM((CHUNK, F2), jnp.float32),
            pltpu.VMEM((B, F2), jnp.float32),
            pltpu.VMEM_SHARED((NS * B, F2), jnp.float32),
        ],
        compiler_params=pltpu.CompilerParams(use_tc_tiling_on_sc=False),
    )(_pool_body)
    return f(batch_idx_flat, f2_rows)


def _head_body(part_ref, bi_ref, wh1_ref, bh1_ref, wh2t_ref, bh2_ref, out_ref):
    x = part_ref[...]
    nr = NW * B // 2
    r = lax.broadcasted_iota(jnp.int32, (B, nr), 1)
    bcol = lax.broadcasted_iota(jnp.int32, (B, nr), 0)
    sel_e = ((2 * r) % B == bcol).astype(jnp.float32)
    sel_o = ((2 * r + 1) % B == bcol).astype(jnp.float32)
    se = jnp.dot(sel_e, x, preferred_element_type=jnp.float32)
    so = jnp.dot(sel_o, x, preferred_element_type=jnp.float32)
    sums = se[:, :F2] + so[:, F2:]
    bi = bi_ref[...]
    counts = [jnp.sum(jnp.where(bi == b, 1.0, 0.0)) for b in range(B)]
    counts = jnp.stack(counts).reshape(B, 1)
    z = sums / jnp.maximum(counts, 1.0)
    h = jnp.dot(z, wh1_ref[...], preferred_element_type=jnp.float32)
    h = jnp.maximum(h + bh1_ref[...], 0.0)
    cn = (((1,), (1,)), ((), ()))
    out_ref[...] = (lax.dot_general(h, wh2t_ref[...], cn,
                                    preferred_element_type=jnp.float32)
                    + bh2_ref[...])


def _head(partials, bi_arr, Wh1, bh1, Wh2, bh2):
    return pl.pallas_call(
        _head_body,
        out_shape=jax.ShapeDtypeStruct((B, 2), jnp.float32),
    )(partials.reshape(NW * B // 2, 128), bi_arr,
      Wh1, bh1.reshape(1, 64), Wh2.T, bh2.reshape(1, 2))


def kernel(coords, feats, W1a, b1a, W1b, b1b, W2, b2, Wh1, bh1, Wh2, bh2):
    f2p, bip, bipre = _encoder(coords, feats, W1a, b1a, W1b, b1b, W2, b2)
    partials = _pool(bipre.reshape(N), f2p)
    return _head(partials, bip, Wh1, bh1, Wh2, bh2)
